# baseline (device time: 76904 ns/iter reference)
import jax
import jax.numpy as jnp
from jax import lax
from jax.experimental import pallas as pl
from jax.experimental.pallas import tpu as pltpu

N_DEV = 4
B, SQ, SKV = 2, 512, 512
HQ, DH = 8, 64
WIN = 128
D_MODEL = 768
D_HEADS = HQ * DH
ROWS = B * SQ
CHUNK = ROWS // (2 * N_DEV)

BF = jnp.bfloat16
F32 = jnp.float32


def _dot(a, b):
    return jnp.dot(a.astype(BF), b.astype(BF), preferred_element_type=F32)


def kernel(x, Wq, K_ext, V_ext, Wo):
    def body(x_ref, wq_ref, k_ref, v_ref, wo_ref, out_ref,
             rsr_ref, rsl_ref, rs_stage_r, rs_stage_l,
             agr_ref, agl_ref, ag_stage_r, ag_stage_l,
             rs_send_r, rs_recv_r, rs_send_l, rs_recv_l,
             ag_send_r, ag_recv_r, ag_send_l, ag_recv_l):
        my = lax.axis_index("i")
        left = lax.rem(my + N_DEV - 1, N_DEV)
        right = lax.rem(my + 1, N_DEV)

        barrier_sem = pltpu.get_barrier_semaphore()
        for nbr in (left, right):
            pl.semaphore_signal(barrier_sem, inc=1, device_id=(nbr,),
                                device_id_type=pl.DeviceIdType.MESH)
        pl.semaphore_wait(barrier_sem, 2)

        def r_off(c):
            return lax.rem(c + 2 * N_DEV, N_DEV) * CHUNK

        def l_off(c):
            return N_DEV * CHUNK + lax.rem(c + 2 * N_DEV, N_DEV) * CHUNK

        wq_my = wq_ref[:, pl.ds(my * D_HEADS, D_HEADS)]
        wo_my = wo_ref[pl.ds(my * D_HEADS, D_HEADS), :]

        def compute_block(b, qs):
            xb = x_ref[b, pl.ds(qs, CHUNK), :]
            qblk = _dot(xb, wq_my)
            qi = qs + lax.broadcasted_iota(jnp.int32, (CHUNK, SKV), 0)
            ki = lax.broadcasted_iota(jnp.int32, (CHUNK, SKV), 1)
            mask = jnp.abs(qi - ki) <= WIN
            cols = []
            for h in range(HQ):
                qbh = qblk[:, h * DH:(h + 1) * DH]
                kbh = k_ref[b, :, h, :]
                vbh = v_ref[b, :, h, :]
                s = _dot(qbh, kbh.T) * 0.125
                s = jnp.where(mask, s, -1e9)
                m = jnp.max(s, axis=-1, keepdims=True)
                w = jnp.exp(s - m)
                w = w / jnp.sum(w, axis=-1, keepdims=True)
                cols.append(_dot(w, vbh))
            ctx = jnp.concatenate(cols, axis=-1)
            return _dot(ctx, wo_my)

        def compute_pair(cr, cl):
            ro, lo = r_off(cr), l_off(cl)
            out_ref[pl.ds(ro, CHUNK), :] = compute_block(0, ro)
            out_ref[pl.ds(lo, CHUNK), :] = compute_block(1, lo - N_DEV * CHUNK)

        compute_pair(my, my)
        for t in range(N_DEV - 1):
            rs_stage_r[...] = out_ref[pl.ds(r_off(my - t), CHUNK), :].astype(BF)
            rs_stage_l[...] = out_ref[pl.ds(l_off(my + t), CHUNK), :].astype(BF)
            rd_r = pltpu.make_async_remote_copy(
                src_ref=rs_stage_r,
                dst_ref=rsr_ref.at[t],
                send_sem=rs_send_r.at[t], recv_sem=rs_recv_r.at[t],
                device_id=(right,), device_id_type=pl.DeviceIdType.MESH,
            )
            rd_l = pltpu.make_async_remote_copy(
                src_ref=rs_stage_l,
                dst_ref=rsl_ref.at[t],
                send_sem=rs_send_l.at[t], recv_sem=rs_recv_l.at[t],
                device_id=(left,), device_id_type=pl.DeviceIdType.MESH,
            )
            rd_r.start()
            rd_l.start()
            compute_pair(my - t - 1, my + t + 1)
            rd_r.wait()
            rd_l.wait()
            out_ref[pl.ds(r_off(my - t - 1), CHUNK), :] += rsr_ref[t].astype(F32)
            out_ref[pl.ds(l_off(my + t + 1), CHUNK), :] += rsl_ref[t].astype(F32)

        ag_stage_r[...] = out_ref[pl.ds(r_off(my + 1), CHUNK), :].astype(BF)
        ag_stage_l[...] = out_ref[pl.ds(l_off(my - 1), CHUNK), :].astype(BF)
        for t in range(N_DEV - 1):
            src_r = ag_stage_r if t == 0 else agr_ref.at[t - 1]
            src_l = ag_stage_l if t == 0 else agl_ref.at[t - 1]
            ag_r = pltpu.make_async_remote_copy(
                src_ref=src_r,
                dst_ref=agr_ref.at[t],
                send_sem=ag_send_r.at[t], recv_sem=ag_recv_r.at[t],
                device_id=(right,), device_id_type=pl.DeviceIdType.MESH,
            )
            ag_l = pltpu.make_async_remote_copy(
                src_ref=src_l,
                dst_ref=agl_ref.at[t],
                send_sem=ag_send_l.at[t], recv_sem=ag_recv_l.at[t],
                device_id=(left,), device_id_type=pl.DeviceIdType.MESH,
            )
            ag_r.start()
            ag_l.start()
            ag_r.wait()
            ag_l.wait()
            out_ref[pl.ds(r_off(my - t), CHUNK), :] = agr_ref[t].astype(F32)
            out_ref[pl.ds(l_off(my + t), CHUNK), :] = agl_ref[t].astype(F32)

    out_flat = pl.pallas_call(
        body,
        out_shape=jax.ShapeDtypeStruct((ROWS, D_MODEL), F32),
        in_specs=[pl.BlockSpec(memory_space=pltpu.VMEM)] * 5,
        out_specs=pl.BlockSpec(memory_space=pltpu.VMEM),
        scratch_shapes=[
            pltpu.VMEM((N_DEV - 1, CHUNK, D_MODEL), BF),
            pltpu.VMEM((N_DEV - 1, CHUNK, D_MODEL), BF),
            pltpu.VMEM((CHUNK, D_MODEL), BF),
            pltpu.VMEM((CHUNK, D_MODEL), BF),
            pltpu.VMEM((N_DEV - 1, CHUNK, D_MODEL), BF),
            pltpu.VMEM((N_DEV - 1, CHUNK, D_MODEL), BF),
            pltpu.VMEM((CHUNK, D_MODEL), BF),
            pltpu.VMEM((CHUNK, D_MODEL), BF),
            pltpu.SemaphoreType.DMA((N_DEV - 1,)),
            pltpu.SemaphoreType.DMA((N_DEV - 1,)),
            pltpu.SemaphoreType.DMA((N_DEV - 1,)),
            pltpu.SemaphoreType.DMA((N_DEV - 1,)),
            pltpu.SemaphoreType.DMA((N_DEV - 1,)),
            pltpu.SemaphoreType.DMA((N_DEV - 1,)),
            pltpu.SemaphoreType.DMA((N_DEV - 1,)),
            pltpu.SemaphoreType.DMA((N_DEV - 1,)),
        ],
        compiler_params=pltpu.CompilerParams(collective_id=0),
    )(x, Wq, K_ext, V_ext, Wo)
    return out_flat.reshape(B, SQ, D_MODEL)
